# Initial kernel scaffold; baseline (speedup 1.0000x reference)
import jax
import jax.numpy as jnp
from jax.experimental import pallas as pl


def kernel(initial_node_states, final_node_states, aux_variables, num_graphs, graph_nodes_list, W_gate, b_gate, W_trans, b_trans, bn_gamma, bn_beta, W1, b1, W2, b2):
    raise NotImplementedError("write your pallas kernel here")


# speedup vs baseline: 3.7094x; 3.7094x over previous
import functools

import jax
import jax.numpy as jnp
from jax import lax
from jax.experimental import pallas as pl
from jax.experimental.pallas import tpu as pltpu
from jax.experimental.pallas import tpu_sc as plsc

N_NODES = 100000
HID = 128
NCLS = 10
CPAD = 16
NW = 32
NPW = 3136
N_PAD = NW * NPW
CHUNK = 112
NCHUNK = NPW // CHUNK
ROWBUF = 448
NGRAPH = 128


def _nodewise_body(init_ref, fin_ref, wgi_ref, wgf_ref, wt_ref, bg_ref, bt_ref,
                   out_ref):
    pid = pl.program_id(0)
    init = init_ref[...]
    fin = fin_ref[...]
    gate = jax.nn.sigmoid(
        jnp.dot(init, wgi_ref[...], preferred_element_type=jnp.float32)
        + jnp.dot(fin, wgf_ref[...], preferred_element_type=jnp.float32)
        + bg_ref[...])
    trans = jnp.dot(fin, wt_ref[...], preferred_element_type=jnp.float32) + bt_ref[...]
    nw = gate * trans
    row = pid * NPW + lax.broadcasted_iota(jnp.int32, (NPW, 1), 0)
    out_ref[...] = jnp.where(row < N_NODES, nw, 0.0)


def _nodewise(init, fin, wgi, wgf, wt, bg, bt):
    grid = (NW,)
    return pl.pallas_call(
        _nodewise_body,
        grid=grid,
        in_specs=[
            pl.BlockSpec((NPW, HID), lambda i: (i, 0)),
            pl.BlockSpec((NPW, HID), lambda i: (i, 0)),
            pl.BlockSpec((HID, CPAD), lambda i: (0, 0)),
            pl.BlockSpec((HID, CPAD), lambda i: (0, 0)),
            pl.BlockSpec((HID, CPAD), lambda i: (0, 0)),
            pl.BlockSpec((1, CPAD), lambda i: (0, 0)),
            pl.BlockSpec((1, CPAD), lambda i: (0, 0)),
        ],
        out_specs=pl.BlockSpec((NPW, CPAD), lambda i: (i, 0)),
        out_shape=jax.ShapeDtypeStruct((N_PAD, CPAD), jnp.float32),
    )(init, fin, wgi, wgf, wt, bg, bt)


def _segsum_body(rows_hbm, ids_hbm, zeros_hbm, out_hbm, ids_v, rows_v, acc_sh):
    c = lax.axis_index("c")
    s = lax.axis_index("s")
    wid = s * 2 + c

    @pl.when(s == 0)
    def _():
        pltpu.sync_copy(zeros_hbm, acc_sh)

    pltpu.sync_copy(ids_hbm.at[wid], ids_v)
    plsc.subcore_barrier()
    for t in range(NPW // ROWBUF):
        pltpu.sync_copy(rows_hbm.at[pl.ds(wid * NPW + t * ROWBUF, ROWBUF)],
                        rows_v)
        for j in range(ROWBUF // CHUNK):
            pltpu.sync_copy(rows_v.at[pl.ds(j * CHUNK, CHUNK)],
                            acc_sh.at[ids_v.at[t * (ROWBUF // CHUNK) + j]],
                            add=True)
    plsc.subcore_barrier()

    @pl.when(s == 0)
    def _():
        pltpu.sync_copy(acc_sh, out_hbm.at[c])


def _segsum(rows, ids2d, zeros):
    mesh = plsc.VectorSubcoreMesh(core_axis_name="c", subcore_axis_name="s",
                                  num_cores=2, num_subcores=16)
    f = pl.kernel(
        _segsum_body,
        out_type=jax.ShapeDtypeStruct((2, NGRAPH, CPAD), jnp.float32),
        mesh=mesh,
        scratch_types=[
            pltpu.VMEM((NCHUNK, CHUNK), jnp.int32),
            pltpu.VMEM((ROWBUF, CPAD), jnp.float32),
            pltpu.VMEM_SHARED((NGRAPH, CPAD), jnp.float32),
        ],
    )
    return f(rows, ids2d, zeros)


def _head_body(p_ref, aux_ref, gg_ref, bgm_ref, ga_ref, bam_ref,
               w1g_ref, w1a_ref, b1_ref, w2_ref, b2_ref, out_ref):
    g = p_ref[0] + p_ref[1]

    def bn(x, gam, bet):
        m = jnp.mean(x, axis=0, keepdims=True)
        d = x - m
        v = jnp.mean(d * d, axis=0, keepdims=True)
        return d / jnp.sqrt(v + 1e-5) * gam + bet

    ng = bn(g, gg_ref[...], bgm_ref[...])
    na = bn(aux_ref[...], ga_ref[...], bam_ref[...])
    h = jnp.maximum(
        jnp.dot(ng, w1g_ref[...], preferred_element_type=jnp.float32)
        + jnp.dot(na, w1a_ref[...], preferred_element_type=jnp.float32)
        + b1_ref[...], 0.0)
    out_ref[...] = (jnp.dot(h, w2_ref[...], preferred_element_type=jnp.float32)
                    + b2_ref[...])


def _head(partials, aux16, gg, bgm, ga, bam, w1g, w1a, b1, w2, b2):
    gx = w1g.shape[1]
    return pl.pallas_call(
        _head_body,
        out_shape=jax.ShapeDtypeStruct((NGRAPH, CPAD), jnp.float32),
    )(partials, aux16, gg, bgm, ga, bam, w1g, w1a, b1, w2, b2)


def kernel(initial_node_states, final_node_states, aux_variables, num_graphs,
           graph_nodes_list, W_gate, b_gate, W_trans, b_trans, bn_gamma,
           bn_beta, W1, b1, W2, b2):
    f32 = jnp.float32
    pad_c = CPAD - NCLS
    wgi = jnp.pad(W_gate[:HID], ((0, 0), (0, pad_c)))
    wgf = jnp.pad(W_gate[HID:], ((0, 0), (0, pad_c)))
    wt = jnp.pad(W_trans, ((0, 0), (0, pad_c)))
    bg = jnp.pad(b_gate, (0, pad_c)).reshape(1, CPAD)
    bt = jnp.pad(b_trans, (0, pad_c)).reshape(1, CPAD)

    nodewise = _nodewise(initial_node_states, final_node_states, wgi, wgf, wt,
                         bg, bt)

    ids_pad = jnp.concatenate([
        graph_nodes_list.astype(jnp.int32),
        jnp.full((N_PAD - N_NODES,), NGRAPH - 1, jnp.int32)])
    ids2d = ids_pad.reshape(NW, NCHUNK, CHUNK)
    zeros = jnp.zeros((NGRAPH, CPAD), f32)
    partials = _segsum(nodewise, ids2d, zeros)

    aux16 = jnp.pad(aux_variables, ((0, 0), (0, CPAD - aux_variables.shape[1])))
    gg = jnp.pad(bn_gamma[:NCLS], (0, pad_c)).reshape(1, CPAD)
    bgm = jnp.pad(bn_beta[:NCLS], (0, pad_c)).reshape(1, CPAD)
    ga = jnp.pad(bn_gamma[NCLS:], (0, CPAD - 2)).reshape(1, CPAD)
    bam = jnp.pad(bn_beta[NCLS:], (0, CPAD - 2)).reshape(1, CPAD)
    gx = W1.shape[1]
    w1g = jnp.pad(W1[:NCLS], ((0, pad_c), (0, 0)))
    w1a = jnp.pad(W1[NCLS:], ((0, CPAD - 2), (0, 0)))
    b1r = b1.reshape(1, gx)
    w2p = jnp.pad(W2, ((0, 0), (0, pad_c)))
    b2r = jnp.pad(b2, (0, pad_c)).reshape(1, CPAD)

    out16 = _head(partials, aux16, gg, bgm, ga, bam, w1g, w1a, b1r, w2p, b2r)
    return out16[:, :NCLS]


# speedup vs baseline: 3.9341x; 1.0606x over previous
import functools

import jax
import jax.numpy as jnp
from jax import lax
from jax.experimental import pallas as pl
from jax.experimental.pallas import tpu as pltpu
from jax.experimental.pallas import tpu_sc as plsc

N_NODES = 100000
HID = 128
NCLS = 10
CPAD = 16
NW = 32
NPW = 3136
N_PAD = NW * NPW
CHUNK = 112
NCHUNK = NPW // CHUNK
ROWBUF = 448
NGRAPH = 128


def _nodewise_body(init_ref, fin_ref, wgi_ref, wgf_ref, wt_ref, bg_ref, bt_ref,
                   out_ref):
    pid = pl.program_id(0)
    init = init_ref[...]
    fin = fin_ref[...]
    gate = jax.nn.sigmoid(
        jnp.dot(init, wgi_ref[...], preferred_element_type=jnp.float32)
        + jnp.dot(fin, wgf_ref[...], preferred_element_type=jnp.float32)
        + bg_ref[...])
    trans = jnp.dot(fin, wt_ref[...], preferred_element_type=jnp.float32) + bt_ref[...]
    nw = gate * trans
    row = pid * NPW + lax.broadcasted_iota(jnp.int32, (NPW, 1), 0)
    out_ref[...] = jnp.where(row < N_NODES, nw, 0.0)


def _nodewise(init, fin, wgi, wgf, wt, bg, bt):
    grid = (NW,)
    return pl.pallas_call(
        _nodewise_body,
        grid=grid,
        in_specs=[
            pl.BlockSpec((NPW, HID), lambda i: (i, 0)),
            pl.BlockSpec((NPW, HID), lambda i: (i, 0)),
            pl.BlockSpec((HID, CPAD), lambda i: (0, 0)),
            pl.BlockSpec((HID, CPAD), lambda i: (0, 0)),
            pl.BlockSpec((HID, CPAD), lambda i: (0, 0)),
            pl.BlockSpec((1, CPAD), lambda i: (0, 0)),
            pl.BlockSpec((1, CPAD), lambda i: (0, 0)),
        ],
        out_specs=pl.BlockSpec((NPW, CPAD), lambda i: (i, 0)),
        out_shape=jax.ShapeDtypeStruct((N_PAD, CPAD), jnp.float32),
    )(init, fin, wgi, wgf, wt, bg, bt)


def _segsum_body(rows_hbm, ids_hbm, zeros_hbm, out_hbm, ids_v, rows_v, acc_sh,
                 lsem, ssem):
    c = lax.axis_index("c")
    s = lax.axis_index("s")
    wid = s * 2 + c
    nt = NPW // ROWBUF
    ns = ROWBUF // CHUNK

    @pl.when(s == 0)
    def _():
        pltpu.sync_copy(zeros_hbm, acc_sh)

    ld = [None, None]
    ld[0] = pltpu.async_copy(rows_hbm.at[pl.ds(wid * NPW, ROWBUF)],
                             rows_v.at[0], lsem)
    pltpu.sync_copy(ids_hbm.at[wid], ids_v)
    plsc.subcore_barrier()
    scats = [[], []]
    for t in range(nt):
        cur = t % 2
        nxt = 1 - cur
        ld[cur].wait()
        if t + 1 < nt:
            for h in scats[nxt]:
                h.wait()
            scats[nxt] = []
            ld[nxt] = pltpu.async_copy(
                rows_hbm.at[pl.ds(wid * NPW + (t + 1) * ROWBUF, ROWBUF)],
                rows_v.at[nxt], lsem)
        for j in range(ns):
            scats[cur].append(pltpu.async_copy(
                rows_v.at[cur, pl.ds(j * CHUNK, CHUNK)],
                acc_sh.at[ids_v.at[t * ns + j]], ssem, add=True))
    for b in range(2):
        for h in scats[b]:
            h.wait()
    plsc.subcore_barrier()

    @pl.when(s == 0)
    def _():
        pltpu.sync_copy(acc_sh, out_hbm.at[c])


def _segsum(rows, ids2d, zeros):
    mesh = plsc.VectorSubcoreMesh(core_axis_name="c", subcore_axis_name="s",
                                  num_cores=2, num_subcores=16)
    f = pl.kernel(
        _segsum_body,
        out_type=jax.ShapeDtypeStruct((2, NGRAPH, CPAD), jnp.float32),
        mesh=mesh,
        scratch_types=[
            pltpu.VMEM((NCHUNK, CHUNK), jnp.int32),
            pltpu.VMEM((2, ROWBUF, CPAD), jnp.float32),
            pltpu.VMEM_SHARED((NGRAPH, CPAD), jnp.float32),
            pltpu.SemaphoreType.DMA,
            pltpu.SemaphoreType.DMA,
        ],
    )
    return f(rows, ids2d, zeros)


def _head_body(p_ref, aux_ref, gg_ref, bgm_ref, ga_ref, bam_ref,
               w1g_ref, w1a_ref, b1_ref, w2_ref, b2_ref, out_ref):
    g = p_ref[0] + p_ref[1]

    def bn(x, gam, bet):
        m = jnp.mean(x, axis=0, keepdims=True)
        d = x - m
        v = jnp.mean(d * d, axis=0, keepdims=True)
        return d / jnp.sqrt(v + 1e-5) * gam + bet

    ng = bn(g, gg_ref[...], bgm_ref[...])
    na = bn(aux_ref[...], ga_ref[...], bam_ref[...])
    h = jnp.maximum(
        jnp.dot(ng, w1g_ref[...], preferred_element_type=jnp.float32)
        + jnp.dot(na, w1a_ref[...], preferred_element_type=jnp.float32)
        + b1_ref[...], 0.0)
    out_ref[...] = (jnp.dot(h, w2_ref[...], preferred_element_type=jnp.float32)
                    + b2_ref[...])


def _head(partials, aux16, gg, bgm, ga, bam, w1g, w1a, b1, w2, b2):
    gx = w1g.shape[1]
    return pl.pallas_call(
        _head_body,
        out_shape=jax.ShapeDtypeStruct((NGRAPH, CPAD), jnp.float32),
    )(partials, aux16, gg, bgm, ga, bam, w1g, w1a, b1, w2, b2)


def kernel(initial_node_states, final_node_states, aux_variables, num_graphs,
           graph_nodes_list, W_gate, b_gate, W_trans, b_trans, bn_gamma,
           bn_beta, W1, b1, W2, b2):
    f32 = jnp.float32
    pad_c = CPAD - NCLS
    wgi = jnp.pad(W_gate[:HID], ((0, 0), (0, pad_c)))
    wgf = jnp.pad(W_gate[HID:], ((0, 0), (0, pad_c)))
    wt = jnp.pad(W_trans, ((0, 0), (0, pad_c)))
    bg = jnp.pad(b_gate, (0, pad_c)).reshape(1, CPAD)
    bt = jnp.pad(b_trans, (0, pad_c)).reshape(1, CPAD)

    nodewise = _nodewise(initial_node_states, final_node_states, wgi, wgf, wt,
                         bg, bt)

    ids_pad = jnp.concatenate([
        graph_nodes_list.astype(jnp.int32),
        jnp.full((N_PAD - N_NODES,), NGRAPH - 1, jnp.int32)])
    ids2d = ids_pad.reshape(NW, NCHUNK, CHUNK)
    zeros = jnp.zeros((NGRAPH, CPAD), f32)
    partials = _segsum(nodewise, ids2d, zeros)

    aux16 = jnp.pad(aux_variables, ((0, 0), (0, CPAD - aux_variables.shape[1])))
    gg = jnp.pad(bn_gamma[:NCLS], (0, pad_c)).reshape(1, CPAD)
    bgm = jnp.pad(bn_beta[:NCLS], (0, pad_c)).reshape(1, CPAD)
    ga = jnp.pad(bn_gamma[NCLS:], (0, CPAD - 2)).reshape(1, CPAD)
    bam = jnp.pad(bn_beta[NCLS:], (0, CPAD - 2)).reshape(1, CPAD)
    gx = W1.shape[1]
    w1g = jnp.pad(W1[:NCLS], ((0, pad_c), (0, 0)))
    w1a = jnp.pad(W1[NCLS:], ((0, CPAD - 2), (0, 0)))
    b1r = b1.reshape(1, gx)
    w2p = jnp.pad(W2, ((0, 0), (0, pad_c)))
    b2r = jnp.pad(b2, (0, pad_c)).reshape(1, CPAD)

    out16 = _head(partials, aux16, gg, bgm, ga, bam, w1g, w1a, b1r, w2p, b2r)
    return out16[:, :NCLS]


# speedup vs baseline: 4.2535x; 1.0812x over previous
import functools

import jax
import jax.numpy as jnp
from jax import lax
from jax.experimental import pallas as pl
from jax.experimental.pallas import tpu as pltpu
from jax.experimental.pallas import tpu_sc as plsc

N_NODES = 100000
HID = 128
NCLS = 10
CPAD = 16
NW = 32
NPW = 3136
N_PAD = NW * NPW
CHUNK = 112
NCHUNK = NPW // CHUNK
ROWBUF = 448
NGRAPH = 128


TB = 12544


def _nodewise_body(init_ref, fin_ref, wgi_ref, wgf_ref, wt_ref, bg_ref, bt_ref,
                   out_ref):
    pid = pl.program_id(0)
    init = init_ref[...]
    fin = fin_ref[...]
    gate = jax.nn.sigmoid(
        jnp.dot(init, wgi_ref[...], preferred_element_type=jnp.float32)
        + jnp.dot(fin, wgf_ref[...], preferred_element_type=jnp.float32)
        + bg_ref[...])
    trans = jnp.dot(fin, wt_ref[...], preferred_element_type=jnp.float32) + bt_ref[...]
    nw = gate * trans
    row = pid * TB + lax.broadcasted_iota(jnp.int32, (TB, 1), 0)
    out_ref[...] = jnp.where(row < N_NODES, nw, 0.0)


def _nodewise(init, fin, wgi, wgf, wt, bg, bt):
    grid = (N_PAD // TB,)
    return pl.pallas_call(
        _nodewise_body,
        grid=grid,
        in_specs=[
            pl.BlockSpec((TB, HID), lambda i: (i, 0)),
            pl.BlockSpec((TB, HID), lambda i: (i, 0)),
            pl.BlockSpec((HID, CPAD), lambda i: (0, 0)),
            pl.BlockSpec((HID, CPAD), lambda i: (0, 0)),
            pl.BlockSpec((HID, CPAD), lambda i: (0, 0)),
            pl.BlockSpec((1, CPAD), lambda i: (0, 0)),
            pl.BlockSpec((1, CPAD), lambda i: (0, 0)),
        ],
        out_specs=pl.BlockSpec((TB, CPAD), lambda i: (i, 0)),
        out_shape=jax.ShapeDtypeStruct((N_PAD, CPAD), jnp.float32),
    )(init, fin, wgi, wgf, wt, bg, bt)


def _segsum_body(rows_hbm, ids_hbm, zeros_hbm, out_hbm, ids_v, rows_v, acc_sh,
                 lsem, ssem):
    c = lax.axis_index("c")
    s = lax.axis_index("s")
    wid = s * 2 + c
    nt = NPW // ROWBUF
    ns = ROWBUF // CHUNK

    @pl.when(s == 0)
    def _():
        pltpu.sync_copy(zeros_hbm, acc_sh)

    ld = [None, None]
    ld[0] = pltpu.async_copy(rows_hbm.at[pl.ds(wid * NPW, ROWBUF)],
                             rows_v.at[0], lsem)
    pltpu.sync_copy(ids_hbm.at[wid], ids_v)
    plsc.subcore_barrier()
    scats = [[], []]
    for t in range(nt):
        cur = t % 2
        nxt = 1 - cur
        ld[cur].wait()
        if t + 1 < nt:
            for h in scats[nxt]:
                h.wait()
            scats[nxt] = []
            ld[nxt] = pltpu.async_copy(
                rows_hbm.at[pl.ds(wid * NPW + (t + 1) * ROWBUF, ROWBUF)],
                rows_v.at[nxt], lsem)
        for j in range(ns):
            scats[cur].append(pltpu.async_copy(
                rows_v.at[cur, pl.ds(j * CHUNK, CHUNK)],
                acc_sh.at[ids_v.at[t * ns + j]], ssem, add=True))
    for b in range(2):
        for h in scats[b]:
            h.wait()
    plsc.subcore_barrier()

    @pl.when(s == 0)
    def _():
        pltpu.sync_copy(acc_sh, out_hbm.at[c])


def _segsum(rows, ids2d, zeros):
    mesh = plsc.VectorSubcoreMesh(core_axis_name="c", subcore_axis_name="s",
                                  num_cores=2, num_subcores=16)
    f = pl.kernel(
        _segsum_body,
        out_type=jax.ShapeDtypeStruct((2, NGRAPH, CPAD), jnp.float32),
        mesh=mesh,
        scratch_types=[
            pltpu.VMEM((NCHUNK, CHUNK), jnp.int32),
            pltpu.VMEM((2, ROWBUF, CPAD), jnp.float32),
            pltpu.VMEM_SHARED((NGRAPH, CPAD), jnp.float32),
            pltpu.SemaphoreType.DMA,
            pltpu.SemaphoreType.DMA,
        ],
    )
    return f(rows, ids2d, zeros)


def _head_body(p_ref, aux_ref, gg_ref, bgm_ref, ga_ref, bam_ref,
               w1g_ref, w1a_ref, b1_ref, w2_ref, b2_ref, out_ref):
    g = p_ref[0] + p_ref[1]

    def bn(x, gam, bet):
        m = jnp.mean(x, axis=0, keepdims=True)
        d = x - m
        v = jnp.mean(d * d, axis=0, keepdims=True)
        return d / jnp.sqrt(v + 1e-5) * gam + bet

    ng = bn(g, gg_ref[...], bgm_ref[...])
    na = bn(aux_ref[...], ga_ref[...], bam_ref[...])
    h = jnp.maximum(
        jnp.dot(ng, w1g_ref[...], preferred_element_type=jnp.float32)
        + jnp.dot(na, w1a_ref[...], preferred_element_type=jnp.float32)
        + b1_ref[...], 0.0)
    out_ref[...] = (jnp.dot(h, w2_ref[...], preferred_element_type=jnp.float32)
                    + b2_ref[...])


def _head(partials, aux16, gg, bgm, ga, bam, w1g, w1a, b1, w2, b2):
    gx = w1g.shape[1]
    return pl.pallas_call(
        _head_body,
        out_shape=jax.ShapeDtypeStruct((NGRAPH, CPAD), jnp.float32),
    )(partials, aux16, gg, bgm, ga, bam, w1g, w1a, b1, w2, b2)


def kernel(initial_node_states, final_node_states, aux_variables, num_graphs,
           graph_nodes_list, W_gate, b_gate, W_trans, b_trans, bn_gamma,
           bn_beta, W1, b1, W2, b2):
    f32 = jnp.float32
    pad_c = CPAD - NCLS
    wgi = jnp.pad(W_gate[:HID], ((0, 0), (0, pad_c)))
    wgf = jnp.pad(W_gate[HID:], ((0, 0), (0, pad_c)))
    wt = jnp.pad(W_trans, ((0, 0), (0, pad_c)))
    bg = jnp.pad(b_gate, (0, pad_c)).reshape(1, CPAD)
    bt = jnp.pad(b_trans, (0, pad_c)).reshape(1, CPAD)

    nodewise = _nodewise(initial_node_states, final_node_states, wgi, wgf, wt,
                         bg, bt)

    ids_pad = jnp.concatenate([
        graph_nodes_list.astype(jnp.int32),
        jnp.full((N_PAD - N_NODES,), NGRAPH - 1, jnp.int32)])
    ids2d = ids_pad.reshape(NW, NCHUNK, CHUNK)
    zeros = jnp.zeros((NGRAPH, CPAD), f32)
    partials = _segsum(nodewise, ids2d, zeros)

    aux16 = jnp.pad(aux_variables, ((0, 0), (0, CPAD - aux_variables.shape[1])))
    gg = jnp.pad(bn_gamma[:NCLS], (0, pad_c)).reshape(1, CPAD)
    bgm = jnp.pad(bn_beta[:NCLS], (0, pad_c)).reshape(1, CPAD)
    ga = jnp.pad(bn_gamma[NCLS:], (0, CPAD - 2)).reshape(1, CPAD)
    bam = jnp.pad(bn_beta[NCLS:], (0, CPAD - 2)).reshape(1, CPAD)
    gx = W1.shape[1]
    w1g = jnp.pad(W1[:NCLS], ((0, pad_c), (0, 0)))
    w1a = jnp.pad(W1[NCLS:], ((0, CPAD - 2), (0, 0)))
    b1r = b1.reshape(1, gx)
    w2p = jnp.pad(W2, ((0, 0), (0, pad_c)))
    b2r = jnp.pad(b2, (0, pad_c)).reshape(1, CPAD)

    out16 = _head(partials, aux16, gg, bgm, ga, bam, w1g, w1a, b1r, w2p, b2r)
    return out16[:, :NCLS]
